# baseline (speedup 1.0000x reference)
import functools

import jax
import jax.numpy as jnp
from jax import lax
from jax.experimental import pallas as pl
from jax.experimental.pallas import tpu as pltpu


BN_EPS = 1e-5
F_IN = 13
H1P = 16
H2P = 8
P_ROWS, P_COLS = 48, 16


def _fused_kernel(x_ref, p_ref, o_ref, hc_ref, s1_ref, q1_ref, s2_ref, q2_ref,
                  *, n_valid, tile_n, masked):
    phase = pl.program_id(0)
    i = pl.program_id(1)
    inv_n = jnp.float32(1.0 / n_valid)

    w1 = p_ref[0:H1P, 0:F_IN]
    w2 = p_ref[16:24, 0:H1P]
    g1 = p_ref[24:40, 1:2]
    be1 = p_ref[24:40, 2:3]
    g2 = p_ref[40:48, 1:2]
    be2 = p_ref[40:48, 2:3]
    w3c = p_ref[40:48, 3:4]
    b3 = p_ref[40:41, 4:5]

    if masked:
        lane = lax.broadcasted_iota(jnp.int32, (H1P, tile_n), 1)
        mask = ((i * tile_n + lane) < n_valid).astype(jnp.float32)
    else:
        mask = None

    def bn_scale_shift(sg, qg, gamma, beta):
        k = sg.shape[0]
        r = lax.broadcasted_iota(jnp.int32, (k, k), 0)
        c = lax.broadcasted_iota(jnp.int32, (k, k), 1)
        q = jnp.sum(jnp.where(r == c, qg, 0.0), axis=1, keepdims=True)
        mean = sg[:, 0:1] * inv_n
        var = jnp.maximum(q * inv_n - mean * mean, 0.0)
        a = gamma * lax.rsqrt(var + BN_EPS)
        return a, beta - mean * a

    @pl.when(jnp.logical_and(phase == 0, i == 0))
    def _init():
        s1_ref[...] = jnp.zeros_like(s1_ref)
        q1_ref[...] = jnp.zeros_like(q1_ref)
        s2_ref[...] = jnp.zeros_like(s2_ref)
        q2_ref[...] = jnp.zeros_like(q2_ref)

    @pl.when(phase == 0)
    def _phase0():
        x_blk = x_ref[...]
        h1 = lax.dot_general(w1, x_blk, (((1,), (0,)), ((), ())),
                             preferred_element_type=jnp.float32)
        h1c = h1.astype(jnp.bfloat16)
        hc_ref[i] = h1c
        hm = (h1 * mask).astype(jnp.bfloat16) if masked else h1c
        ones16 = jnp.ones((H1P, tile_n), jnp.bfloat16)
        s1_ref[...] += lax.dot_general(hm, ones16, (((1,), (1,)), ((), ())),
                                       preferred_element_type=jnp.float32)
        q1_ref[...] += lax.dot_general(hm, hm, (((1,), (1,)), ((), ())),
                                       preferred_element_type=jnp.float32)

    @pl.when(phase == 1)
    def _phase1():
        a1, c1 = bn_scale_shift(s1_ref[...], q1_ref[...], g1, be1)
        h1 = hc_ref[i].astype(jnp.float32)
        h1a16 = jnp.maximum(h1 * a1 + c1, 0.0).astype(jnp.bfloat16)
        h2 = lax.dot_general(w2.astype(jnp.bfloat16), h1a16,
                             (((1,), (0,)), ((), ())),
                             preferred_element_type=jnp.float32)
        h2c = h2.astype(jnp.bfloat16)
        hc_ref[i, 0:H2P, :] = h2c
        hm2 = (h2 * mask[0:H2P, :]).astype(jnp.bfloat16) if masked else h2c
        ones8 = jnp.ones((H2P, tile_n), jnp.bfloat16)
        s2_ref[...] += lax.dot_general(hm2, ones8, (((1,), (1,)), ((), ())),
                                       preferred_element_type=jnp.float32)
        q2_ref[...] += lax.dot_general(hm2, hm2, (((1,), (1,)), ((), ())),
                                       preferred_element_type=jnp.float32)

    @pl.when(phase == 2)
    def _phase2():
        h2 = hc_ref[i, 0:H2P, :].astype(jnp.float32)
        a2, c2 = bn_scale_shift(s2_ref[...], q2_ref[...], g2, be2)
        h2a = jnp.maximum(h2 * a2 + c2, 0.0)
        h3 = lax.dot_general(w3c, h2a, (((0,), (0,)), ((), ())),
                             preferred_element_type=jnp.float32) + b3
        o_ref[...] = jax.nn.sigmoid(h3)


def _round_up(a: int, b: int) -> int:
    return (a + b - 1) // b * b


def _forward(x, packed_params, *, tile_n=131072):
    n, f = x.shape
    assert f == F_IN, f

    if n <= tile_n:
        tile = _round_up(max(n, 1), 8)
    else:
        tile = _round_up(tile_n, 128)
    padded_n = _round_up(n, tile)
    if padded_n != n:
        x = jnp.pad(x, ((0, padded_n - n), (0, 0)))
    num_tiles = padded_n // tile
    last = num_tiles - 1
    xt = x.T

    out = pl.pallas_call(
        functools.partial(_fused_kernel, n_valid=n, tile_n=tile,
                          masked=padded_n != n),
        out_shape=jax.ShapeDtypeStruct((1, padded_n), jnp.float32),
        grid=(3, num_tiles),
        in_specs=[
            pl.BlockSpec((F_IN, tile),
                         lambda p, i: (0, jnp.where(p == 0, i, last))),
            pl.BlockSpec((P_ROWS, P_COLS), lambda p, i: (0, 0)),
        ],
        out_specs=pl.BlockSpec((1, tile),
                               lambda p, i: (0, jnp.where(p == 2, i, 0))),
        scratch_shapes=[
            pltpu.VMEM((num_tiles, H1P, tile), jnp.bfloat16),
            pltpu.VMEM((H1P, H1P), jnp.float32),
            pltpu.VMEM((H1P, H1P), jnp.float32),
            pltpu.VMEM((H2P, H2P), jnp.float32),
            pltpu.VMEM((H2P, H2P), jnp.float32),
        ],
        compiler_params=pltpu.CompilerParams(
            dimension_semantics=("arbitrary", "arbitrary"),
            vmem_limit_bytes=56 * 1024 * 1024,
        ),
    )(xt, packed_params)

    return out[:, :n].T


def kernel(x, packed_params):
    return _forward(x, packed_params)


# speedup vs baseline: 1.4458x; 1.4458x over previous
import functools

import jax
import jax.numpy as jnp
from jax import lax
from jax.experimental import pallas as pl
from jax.experimental.pallas import tpu as pltpu


BN_EPS = 1e-5
F_IN = 13
H1P = 16
H2P = 8
P_ROWS, P_COLS = 48, 16


def _fused_kernel(x_ref, p_ref, o_ref, hc_ref, s1_ref, q1_ref, s2_ref, q2_ref,
                  *, n_valid, tile_n, masked):
    phase = pl.program_id(0)
    i = pl.program_id(1)
    inv_n = jnp.float32(1.0 / n_valid)

    w1 = p_ref[0:H1P, 0:F_IN]
    w2 = p_ref[16:24, 0:H1P]
    g1 = p_ref[24:40, 1:2]
    be1 = p_ref[24:40, 2:3]
    g2 = p_ref[40:48, 1:2]
    be2 = p_ref[40:48, 2:3]
    w3c = p_ref[40:48, 3:4]
    b3 = p_ref[40:41, 4:5]

    if masked:
        lane = lax.broadcasted_iota(jnp.int32, (H1P, tile_n), 1)
        mask = ((i * tile_n + lane) < n_valid).astype(jnp.float32)
    else:
        mask = None

    def bn_scale_shift(s, q, gamma, beta):
        mean = s * inv_n
        var = jnp.maximum(q * inv_n - mean * mean, 0.0)
        a = gamma * lax.rsqrt(var + BN_EPS)
        return a, beta - mean * a

    @pl.when(jnp.logical_and(phase == 0, i == 0))
    def _init():
        s1_ref[...] = jnp.zeros_like(s1_ref)
        q1_ref[...] = jnp.zeros_like(q1_ref)
        s2_ref[...] = jnp.zeros_like(s2_ref)
        q2_ref[...] = jnp.zeros_like(q2_ref)

    @pl.when(phase == 0)
    def _phase0():
        x_blk = x_ref[...]
        h1 = lax.dot_general(w1, x_blk, (((1,), (0,)), ((), ())),
                             preferred_element_type=jnp.float32)
        hc_ref[i] = h1.astype(jnp.bfloat16)
        hm = h1 * mask if masked else h1
        s1_ref[...] += jnp.sum(hm, axis=-1, keepdims=True)
        q1_ref[...] += jnp.sum(hm * h1, axis=-1, keepdims=True)

    @pl.when(phase == 1)
    def _phase1():
        a1, c1 = bn_scale_shift(s1_ref[...], q1_ref[...], g1, be1)
        h1 = hc_ref[i].astype(jnp.float32)
        h1a16 = jnp.maximum(h1 * a1 + c1, 0.0).astype(jnp.bfloat16)
        h2 = lax.dot_general(w2.astype(jnp.bfloat16), h1a16,
                             (((1,), (0,)), ((), ())),
                             preferred_element_type=jnp.float32)
        hc_ref[i, 0:H2P, :] = h2.astype(jnp.bfloat16)
        hm2 = h2 * mask[0:H2P, :] if masked else h2
        s2_ref[...] += jnp.sum(hm2, axis=-1, keepdims=True)
        q2_ref[...] += jnp.sum(hm2 * h2, axis=-1, keepdims=True)

    @pl.when(phase == 2)
    def _phase2():
        h2 = hc_ref[i, 0:H2P, :].astype(jnp.float32)
        a2, c2 = bn_scale_shift(s2_ref[...], q2_ref[...], g2, be2)
        h2a = jnp.maximum(h2 * a2 + c2, 0.0)
        h3 = lax.dot_general(w3c, h2a, (((0,), (0,)), ((), ())),
                             preferred_element_type=jnp.float32) + b3
        o_ref[...] = jax.nn.sigmoid(h3)


def _round_up(a: int, b: int) -> int:
    return (a + b - 1) // b * b


def _forward(x, packed_params, *, tile_n=131072):
    n, f = x.shape
    assert f == F_IN, f

    if n <= tile_n:
        tile = _round_up(max(n, 1), 8)
    else:
        tile = _round_up(tile_n, 128)
    padded_n = _round_up(n, tile)
    if padded_n != n:
        x = jnp.pad(x, ((0, padded_n - n), (0, 0)))
    num_tiles = padded_n // tile
    last = num_tiles - 1
    xt = x.T

    out = pl.pallas_call(
        functools.partial(_fused_kernel, n_valid=n, tile_n=tile,
                          masked=padded_n != n),
        out_shape=jax.ShapeDtypeStruct((1, padded_n), jnp.float32),
        grid=(3, num_tiles),
        in_specs=[
            pl.BlockSpec((F_IN, tile),
                         lambda p, i: (0, jnp.where(p == 0, i, last))),
            pl.BlockSpec((P_ROWS, P_COLS), lambda p, i: (0, 0)),
        ],
        out_specs=pl.BlockSpec((1, tile),
                               lambda p, i: (0, jnp.where(p == 2, i, 0))),
        scratch_shapes=[
            pltpu.VMEM((num_tiles, H1P, tile), jnp.bfloat16),
            pltpu.VMEM((H1P, 1), jnp.float32),
            pltpu.VMEM((H1P, 1), jnp.float32),
            pltpu.VMEM((H2P, 1), jnp.float32),
            pltpu.VMEM((H2P, 1), jnp.float32),
        ],
        compiler_params=pltpu.CompilerParams(
            dimension_semantics=("arbitrary", "arbitrary"),
            vmem_limit_bytes=56 * 1024 * 1024,
        ),
    )(xt, packed_params)

    return out[:, :n].T


def kernel(x, packed_params):
    return _forward(x, packed_params)
